# Initial kernel scaffold; baseline (speedup 1.0000x reference)
import jax
import jax.numpy as jnp
from jax.experimental import pallas as pl


def kernel(inputs, W_dense, W_q0, W_q1, W_q2, Wp, v):
    raise NotImplementedError("write your pallas kernel here")


# speedup vs baseline: 2.1247x; 2.1247x over previous
import functools

import jax
import jax.numpy as jnp
from jax.experimental import pallas as pl
from jax.experimental.pallas import tpu as pltpu

LARGE_NUMBER = 100000000.0
_CHUNK = 8


def _decoder_body(S, B, H, att, qdim,
                  tin_ref, wdt_ref, w0t_ref, w1t_ref, w2t_ref, wpt_ref,
                  v_ref, g_ref,
                  tour_ref, lp_ref, ent_ref,
                  e_ref, mask_ref, scores_ref, r_ref):
    f32 = jnp.float32
    nchunk = S // _CHUNK

    def fill_e(c, _):
        x = tin_ref[pl.ds(c * _CHUNK, _CHUNK)]
        x2 = x.reshape(_CHUNK * B, H)
        e2 = jnp.dot(x2, wdt_ref[:], preferred_element_type=f32)
        e_ref[pl.ds(c * _CHUNK, _CHUNK)] = e2.reshape(_CHUNK, B, att)
        return 0

    jax.lax.fori_loop(0, nchunk, fill_e, 0)

    mask_ref[:] = jnp.zeros((S, B), dtype=f32)
    r_ref[:] = jnp.zeros((3, B, H), dtype=f32)

    iota_s = jax.lax.broadcasted_iota(jnp.int32, (S, B), 0)
    v_row = v_ref[:]

    def step(t, carry):
        lp, ent = carry

        rA = r_ref[jax.lax.rem(t, 3)]
        rB = r_ref[jax.lax.rem(t + 1, 3)]
        rC = r_ref[jax.lax.rem(t + 2, 3)]
        d0 = jnp.dot(rA, w0t_ref[:], preferred_element_type=f32)
        d1 = jnp.dot(rB, w1t_ref[:], preferred_element_type=f32)
        d2 = jnp.dot(rC, w2t_ref[:], preferred_element_type=f32)
        query = jnp.maximum(d0 + d1 + d2, 0.0)
        eq = jnp.dot(query, wpt_ref[:], preferred_element_type=f32)

        def score_chunk(c, _):
            ec = e_ref[pl.ds(c * _CHUNK, _CHUNK)]
            x = jnp.tanh(ec + eq[None, :, :])
            scores_ref[pl.ds(c * _CHUNK, _CHUNK)] = jnp.sum(
                v_row[None, :, :] * x, axis=-1)
            return 0

        jax.lax.fori_loop(0, nchunk, score_chunk, 0)

        scores = 10.0 * jnp.tanh(scores_ref[:])
        masked = jnp.clip(scores - LARGE_NUMBER * mask_ref[:],
                          -LARGE_NUMBER, LARGE_NUMBER)

        z = masked + g_ref[t]
        zmax = jnp.max(z, axis=0)
        idx = jnp.min(jnp.where(z == zmax[None, :], iota_s, S), axis=0)
        oh = (iota_s == idx[None, :]).astype(f32)

        xmax = jnp.max(masked, axis=0)
        sh = masked - xmax[None, :]
        logz = jnp.log(jnp.sum(jnp.exp(sh), axis=0))
        chosen = jnp.sum(oh * sh, axis=0)
        lp = lp + (chosen - logz)[None, :]
        logp = sh - logz[None, :]
        probs = jnp.exp(logp)
        ent = ent - jnp.sum(probs * logp, axis=0)[None, :]

        tour_ref[pl.ds(t, 1)] = idx[None, :]
        mask_ref[:] = mask_ref[:] + oh

        iota_c = jax.lax.broadcasted_iota(jnp.int32, (_CHUNK, B), 0)

        def gather_chunk(c, acc):
            xc = tin_ref[pl.ds(c * _CHUNK, _CHUNK)]
            ohc = ((iota_c + c * _CHUNK) == idx[None, :]).astype(f32)
            return acc + jnp.sum(ohc[:, :, None] * xc, axis=0)

        r_new = jax.lax.fori_loop(0, nchunk, gather_chunk,
                                  jnp.zeros((B, H), dtype=f32))
        r_ref[jax.lax.rem(t, 3)] = r_new
        return (lp, ent)

    lp0 = jnp.zeros((1, B), dtype=f32)
    ent0 = jnp.zeros((1, B), dtype=f32)
    lp, ent = jax.lax.fori_loop(0, S, step, (lp0, ent0))
    lp_ref[:] = lp
    ent_ref[:] = ent


def kernel(inputs, W_dense, W_q0, W_q1, W_q2, Wp, v):
    B, S, H = inputs.shape
    att = W_dense.shape[0]
    qdim = W_q0.shape[0]
    f32 = jnp.float32

    tin = jnp.transpose(inputs, (1, 0, 2))
    wdt = W_dense.T
    w0t, w1t, w2t = W_q0.T, W_q1.T, W_q2.T
    wpt = Wp.T
    v_row = v.reshape(1, att)

    skey = jax.random.key(42)
    keys = jax.vmap(lambda t: jax.random.fold_in(skey, t))(jnp.arange(S))
    g = jax.vmap(lambda k: jax.random.gumbel(k, (B, S), f32))(keys)
    g = jnp.transpose(g, (0, 2, 1))

    body = functools.partial(_decoder_body, S, B, H, att, qdim)
    tour_steps, lp, ent = pl.pallas_call(
        body,
        out_shape=(
            jax.ShapeDtypeStruct((S, B), jnp.int32),
            jax.ShapeDtypeStruct((1, B), f32),
            jax.ShapeDtypeStruct((1, B), f32),
        ),
        scratch_shapes=[
            pltpu.VMEM((S, B, att), f32),
            pltpu.VMEM((S, B), f32),
            pltpu.VMEM((S, B), f32),
            pltpu.VMEM((3, B, H), f32),
        ],
        compiler_params=pltpu.CompilerParams(
            vmem_limit_bytes=128 * 1024 * 1024),
    )(tin, wdt, w0t, w1t, w2t, wpt, v_row, g)

    steps_t = tour_steps.T
    tour = jnp.concatenate([steps_t, steps_t[:, :1]], axis=1)
    return (tour, lp[0], ent[0])


# speedup vs baseline: 3.3525x; 1.5779x over previous
import functools

import jax
import jax.numpy as jnp
from jax.experimental import pallas as pl
from jax.experimental.pallas import tpu as pltpu

LARGE_NUMBER = 100000000.0
_CHUNK = 16


def _decoder_body(S, B, H, att, qdim,
                  tin_ref, wdt_ref, w0t_ref, w1t_ref, w2t_ref, wpt_ref,
                  v_ref, g_ref,
                  tour_ref, lp_ref, ent_ref,
                  e_ref, mask_ref, scores_ref, r_ref,
                  idx_vref, idx_sref, dma_sem):
    f32 = jnp.float32
    nchunk = S // _CHUNK

    def fill_e(c, _):
        x = tin_ref[pl.ds(c * _CHUNK, _CHUNK)]
        x2 = x.reshape(_CHUNK * B, H)
        e2 = jnp.dot(x2, wdt_ref[:], preferred_element_type=f32)
        e_ref[pl.ds(c * _CHUNK, _CHUNK)] = e2.reshape(_CHUNK, B, att)
        return 0

    jax.lax.fori_loop(0, nchunk, fill_e, 0)

    mask_ref[:] = jnp.zeros((S, B), dtype=f32)
    r_ref[:] = jnp.zeros((3, B, H), dtype=f32)

    iota_s = jax.lax.broadcasted_iota(jnp.int32, (S, B), 0)
    v_row = v_ref[:]

    def step(t, carry):
        lp, ent = carry

        rA = r_ref[jax.lax.rem(t, 3)]
        rB = r_ref[jax.lax.rem(t + 1, 3)]
        rC = r_ref[jax.lax.rem(t + 2, 3)]
        d0 = jnp.dot(rA, w0t_ref[:], preferred_element_type=f32)
        d1 = jnp.dot(rB, w1t_ref[:], preferred_element_type=f32)
        d2 = jnp.dot(rC, w2t_ref[:], preferred_element_type=f32)
        query = jnp.maximum(d0 + d1 + d2, 0.0)
        eq = jnp.dot(query, wpt_ref[:], preferred_element_type=f32)

        def score_chunk(c, _):
            ec = e_ref[pl.ds(c * _CHUNK, _CHUNK)]
            x = jnp.tanh(ec + eq[None, :, :])
            scores_ref[pl.ds(c * _CHUNK, _CHUNK)] = jnp.sum(
                v_row[None, :, :] * x, axis=-1)
            return 0

        jax.lax.fori_loop(0, nchunk, score_chunk, 0)

        scores = 10.0 * jnp.tanh(scores_ref[:])
        masked = jnp.clip(scores - LARGE_NUMBER * mask_ref[:],
                          -LARGE_NUMBER, LARGE_NUMBER)

        z = masked + g_ref[t]
        zmax = jnp.max(z, axis=0)
        idx = jnp.min(jnp.where(z == zmax[None, :], iota_s, S), axis=0)
        oh = (iota_s == idx[None, :]).astype(f32)

        xmax = jnp.max(masked, axis=0)
        sh = masked - xmax[None, :]
        logz = jnp.log(jnp.sum(jnp.exp(sh), axis=0))
        chosen = jnp.sum(oh * sh, axis=0)
        lp = lp + (chosen - logz)[None, :]
        logp = sh - logz[None, :]
        probs = jnp.exp(logp)
        ent = ent - jnp.sum(probs * logp, axis=0)[None, :]

        tour_ref[pl.ds(t, 1)] = idx[None, :]
        mask_ref[:] = mask_ref[:] + oh

        idx_vref[:] = idx[None, :]
        cp = pltpu.make_async_copy(idx_vref, idx_sref, dma_sem)
        cp.start()
        cp.wait()
        slot = jax.lax.rem(t, 3)

        def gather_b(b, _):
            iv = idx_sref[0, b]
            r_ref[pl.ds(slot, 1), pl.ds(b, 1), :] = (
                tin_ref[pl.ds(iv, 1), pl.ds(b, 1), :])
            return 0

        jax.lax.fori_loop(0, B, gather_b, 0)
        return (lp, ent)

    lp0 = jnp.zeros((1, B), dtype=f32)
    ent0 = jnp.zeros((1, B), dtype=f32)
    lp, ent = jax.lax.fori_loop(0, S, step, (lp0, ent0))
    lp_ref[:] = lp
    ent_ref[:] = ent


def kernel(inputs, W_dense, W_q0, W_q1, W_q2, Wp, v):
    B, S, H = inputs.shape
    att = W_dense.shape[0]
    qdim = W_q0.shape[0]
    f32 = jnp.float32

    tin = jnp.transpose(inputs, (1, 0, 2))
    wdt = W_dense.T
    w0t, w1t, w2t = W_q0.T, W_q1.T, W_q2.T
    wpt = Wp.T
    v_row = v.reshape(1, att)

    skey = jax.random.key(42)
    keys = jax.vmap(lambda t: jax.random.fold_in(skey, t))(jnp.arange(S))
    g = jax.vmap(lambda k: jax.random.gumbel(k, (B, S), f32))(keys)
    g = jnp.transpose(g, (0, 2, 1))

    body = functools.partial(_decoder_body, S, B, H, att, qdim)
    tour_steps, lp, ent = pl.pallas_call(
        body,
        out_shape=(
            jax.ShapeDtypeStruct((S, B), jnp.int32),
            jax.ShapeDtypeStruct((1, B), f32),
            jax.ShapeDtypeStruct((1, B), f32),
        ),
        scratch_shapes=[
            pltpu.VMEM((S, B, att), f32),
            pltpu.VMEM((S, B), f32),
            pltpu.VMEM((S, B), f32),
            pltpu.VMEM((3, B, H), f32),
            pltpu.VMEM((1, B), jnp.int32),
            pltpu.SMEM((1, B), jnp.int32),
            pltpu.SemaphoreType.DMA,
        ],
        compiler_params=pltpu.CompilerParams(
            vmem_limit_bytes=128 * 1024 * 1024),
    )(tin, wdt, w0t, w1t, w2t, wpt, v_row, g)

    steps_t = tour_steps.T
    tour = jnp.concatenate([steps_t, steps_t[:, :1]], axis=1)
    return (tour, lp[0], ent[0])
